# baseline (speedup 1.0000x reference)
import jax
import jax.numpy as jnp
from jax import lax
from jax.experimental import pallas as pl
from jax.experimental.pallas import tpu as pltpu
from jax.experimental.pallas import tpu_sc as plsc

NC, NS = 2, 16
NW = NC * NS
CB = 128
NJ = 200
NG = 3
NO = 2
BATCH = 4096
HIST = 200
DIM = 64
L = 16


def _body(table2_hbm, xt_hbm, out_hbm, idx_v, pidx_v, rows_v, outb_v, gsems, wsems):
    wid = lax.axis_index("s") * NC + lax.axis_index("c")
    b0 = wid * CB

    pltpu.sync_copy(xt_hbm.at[:, pl.ds(b0, CB)], idx_v)

    iotas = [lax.iota(jnp.int32, L) + (L * g) for g in range(8)]

    def start_gather(h, s):
        for g in range(8):
            pidx_v[s, pl.ds(L * g, L)] = lax.shift_right_logical(
                idx_v[h, pl.ds(L * g, L)], 1
            )
        pltpu.async_copy(
            table2_hbm.at[pidx_v.at[s]], rows_v.at[s], gsems.at[s]
        )

    def wait_gather(s):
        pltpu.make_async_copy(
            table2_hbm.at[pidx_v.at[s]], rows_v.at[s], gsems.at[s]
        ).wait()

    def start_write(h, o):
        pltpu.async_copy(
            outb_v.at[o], out_hbm.at[h, :, pl.ds(b0, CB)], wsems.at[o]
        )

    def wait_write(o):
        pltpu.make_async_copy(
            outb_v.at[o], out_hbm.at[0, :, pl.ds(b0, CB)], wsems.at[o]
        ).wait()

    def transpose_select(h, s, o):
        rows = rows_v.at[s]
        outb = outb_v.at[o]
        hvs = [
            lax.shift_left(
                lax.bitwise_and(idx_v[h, pl.ds(L * g, L)], 1), 6
            )
            for g in range(8)
        ]

        @plsc.parallel_loop(0, DIM, unroll=16)
        def _(d):
            for g in range(8):
                val = plsc.load_gather(rows, [iotas[g], hvs[g] + d])
                outb[d, pl.ds(L * g, L)] = val

    def step(h, first, mid, last):
        s = h % NG
        if not last:
            start_gather(h + NG - 1, (h - 1) % NG)
        wait_gather(s)
        o = h % NO
        if not first:
            wait_write(o)
        transpose_select(h, s, o)
        start_write(h, o)

    for s in range(NG - 1):
        start_gather(s, s)

    step(0, first=True, mid=False, last=False)
    step(1, first=True, mid=False, last=False)

    @pl.loop(2, NJ - NG + 1)
    def _(h):
        step(h, first=False, mid=True, last=False)

    for h in range(NJ - NG + 1, NJ):
        step(h, first=False, mid=False, last=True)

    for o in range(NO):
        wait_write(o)


_gather = pl.kernel(
    _body,
    out_type=jax.ShapeDtypeStruct((HIST, DIM, BATCH), jnp.float32),
    mesh=plsc.VectorSubcoreMesh(
        core_axis_name="c", subcore_axis_name="s", num_cores=NC, num_subcores=NS
    ),
    scratch_types=[
        pltpu.VMEM((NJ, CB), jnp.int32),
        pltpu.VMEM((NG, CB), jnp.int32),
        pltpu.VMEM((NG, CB, 128), jnp.float32),
        pltpu.VMEM((NO, DIM, CB), jnp.float32),
        pltpu.SemaphoreType.DMA((NG,)),
        pltpu.SemaphoreType.DMA((NO,)),
    ],
    compiler_params=pltpu.CompilerParams(needs_layout_passes=False),
)


def kernel(x, table):
    table2 = table.reshape(500000, 128)
    xt = x.T
    outt = _gather(table2, xt)
    return outt.transpose(2, 0, 1)


# speedup vs baseline: 1.5213x; 1.5213x over previous
import jax
import jax.numpy as jnp
from jax import lax
from jax.experimental import pallas as pl
from jax.experimental.pallas import tpu as pltpu
from jax.experimental.pallas import tpu_sc as plsc

NC, NS = 2, 16
NW = NC * NS
CB = 128
NJ = 200
NG = 3
NO = 2
BATCH = 4096
HIST = 200
DIM = 64
L = 16


def _body(table2_hbm, xt_hbm, out_hbm, idx_v, pidx_v, rows_v, outb_v, gsems, wsems):
    wid = lax.axis_index("s") * NC + lax.axis_index("c")
    b0 = wid * CB

    pltpu.sync_copy(xt_hbm.at[:, pl.ds(b0, CB)], idx_v)

    iota = lax.iota(jnp.int32, L)
    iotas = [iota + (L * g) for g in range(8)]

    def start_gather(h, s):
        for g in range(8):
            pidx_v[s, pl.ds(L * g, L)] = lax.shift_right_logical(
                idx_v[h, pl.ds(L * g, L)], 1
            )
        pltpu.async_copy(
            table2_hbm.at[pidx_v.at[s]], rows_v.at[s], gsems.at[s]
        )

    def wait_gather(s):
        pltpu.make_async_copy(
            table2_hbm.at[pidx_v.at[s]], rows_v.at[s], gsems.at[s]
        ).wait()

    def start_write(h, o):
        pltpu.async_copy(
            outb_v.at[o], out_hbm.at[h, :, pl.ds(b0, CB)], wsems.at[o]
        )

    def wait_write(o):
        pltpu.make_async_copy(
            outb_v.at[o], out_hbm.at[0, :, pl.ds(b0, CB)], wsems.at[o]
        ).wait()

    def transpose_select(h, s, o):
        rows = rows_v.at[s]
        outb = outb_v.at[o]
        hvs = [
            lax.shift_left(
                lax.bitwise_and(idx_v[h, pl.ds(L * g, L)], 1), 6
            )
            for g in range(8)
        ]

        @plsc.parallel_loop(0, DIM, unroll=8)
        def _(d):
            w = lax.bitwise_and(d + iota, 63)
            for g in range(8):
                val = plsc.load_gather(rows, [iotas[g], hvs[g] + w])
                plsc.store_scatter(outb, [w, iotas[g]], val)

    def step(h, first, mid, last):
        s = h % NG
        if not last:
            start_gather(h + NG - 1, (h - 1) % NG)
        wait_gather(s)
        o = h % NO
        if not first:
            wait_write(o)
        transpose_select(h, s, o)
        start_write(h, o)

    for s in range(NG - 1):
        start_gather(s, s)

    step(0, first=True, mid=False, last=False)
    step(1, first=True, mid=False, last=False)

    @pl.loop(2, NJ - NG + 1)
    def _(h):
        step(h, first=False, mid=True, last=False)

    for h in range(NJ - NG + 1, NJ):
        step(h, first=False, mid=False, last=True)

    for o in range(NO):
        wait_write(o)


_gather = pl.kernel(
    _body,
    out_type=jax.ShapeDtypeStruct((HIST, DIM, BATCH), jnp.float32),
    mesh=plsc.VectorSubcoreMesh(
        core_axis_name="c", subcore_axis_name="s", num_cores=NC, num_subcores=NS
    ),
    scratch_types=[
        pltpu.VMEM((NJ, CB), jnp.int32),
        pltpu.VMEM((NG, CB), jnp.int32),
        pltpu.VMEM((NG, CB, 128), jnp.float32),
        pltpu.VMEM((NO, DIM, CB), jnp.float32),
        pltpu.SemaphoreType.DMA((NG,)),
        pltpu.SemaphoreType.DMA((NO,)),
    ],
    compiler_params=pltpu.CompilerParams(needs_layout_passes=False),
)


def kernel(x, table):
    table2 = table.reshape(500000, 128)
    xt = x.T
    outt = _gather(table2, xt)
    return outt.transpose(2, 0, 1)


# speedup vs baseline: 2.9261x; 1.9235x over previous
import jax
import jax.numpy as jnp
from jax import lax
from jax.experimental import pallas as pl
from jax.experimental.pallas import tpu as pltpu
from jax.experimental.pallas import tpu_sc as plsc

NC, NS = 2, 16
NW = NC * NS
CB = 128
NJ = 200
NG = 3
NO = 2
BATCH = 4096
HIST = 200
DIM = 64
L = 16


def _body(table2_hbm, xt_hbm, out_hbm, idx_v, pidx_v, rows_v, outb_v, gsems, wsems):
    wid = lax.axis_index("s") * NC + lax.axis_index("c")
    b0 = wid * CB

    pltpu.sync_copy(xt_hbm.at[:, pl.ds(b0, CB)], idx_v)

    iota = lax.iota(jnp.int32, L)
    iotas = [iota + (L * g) for g in range(8)]

    def start_gather(h, s):
        for g in range(8):
            pidx_v[s, pl.ds(L * g, L)] = lax.shift_right_logical(
                idx_v[h, pl.ds(L * g, L)], 1
            )
        pltpu.async_copy(
            table2_hbm.at[pidx_v.at[s]], rows_v.at[s], gsems.at[s]
        )

    def wait_gather(s):
        pltpu.make_async_copy(
            table2_hbm.at[pidx_v.at[s]], rows_v.at[s], gsems.at[s]
        ).wait()

    def start_write(h, o):
        pltpu.async_copy(
            outb_v.at[o], out_hbm.at[h, :, pl.ds(b0, CB)], wsems.at[o]
        )

    def wait_write(o):
        pltpu.make_async_copy(
            outb_v.at[o], out_hbm.at[0, :, pl.ds(b0, CB)], wsems.at[o]
        ).wait()

    def transpose_select(h, s, o):
        rows = rows_v.at[s]
        outb = outb_v.at[o]
        hvs = [
            lax.shift_left(
                lax.bitwise_and(idx_v[h, pl.ds(L * g, L)], 1), 6
            )
            for g in range(8)
        ]

        @plsc.parallel_loop(0, DIM, unroll=8)
        def _(d):
            w = lax.bitwise_and(d + iota, 63)
            for g in range(8):
                val = plsc.load_gather(rows, [iotas[g], hvs[g] + w])
                plsc.store_scatter(outb, [w, iotas[g]], val)

    def step(h, first, mid, last):
        s = h % NG
        if not last:
            start_gather(h + NG - 1, (h - 1) % NG)
        wait_gather(s)
        o = h % NO
        if not first:
            wait_write(o)
        transpose_select(h, s, o)
        start_write(h, o)

    for s in range(NG - 1):
        start_gather(s, s)

    step(0, first=True, mid=False, last=False)
    step(1, first=True, mid=False, last=False)

    @pl.loop(2, NJ - NG + 1)
    def _(h):
        step(h, first=False, mid=True, last=False)

    for h in range(NJ - NG + 1, NJ):
        step(h, first=False, mid=False, last=True)

    for o in range(NO):
        wait_write(o)


_gather = pl.kernel(
    _body,
    out_type=jax.ShapeDtypeStruct((HIST, DIM, BATCH), jnp.float32),
    mesh=plsc.VectorSubcoreMesh(
        core_axis_name="c", subcore_axis_name="s", num_cores=NC, num_subcores=NS
    ),
    scratch_types=[
        pltpu.VMEM((NJ, CB), jnp.int32),
        pltpu.VMEM((NG, CB), jnp.int32),
        pltpu.VMEM((NG, CB, 128), jnp.float32),
        pltpu.VMEM((NO, DIM, CB), jnp.float32),
        pltpu.SemaphoreType.DMA((NG,)),
        pltpu.SemaphoreType.DMA((NO,)),
    ],
    compiler_params=pltpu.CompilerParams(needs_layout_passes=False),
)


NCHUNK = 122
NMAIN = NCHUNK * NW


def _repack_body(tt_hbm, t2_hbm, in_v, out_v, tin_v, tout_v, isems, osems):
    wid = lax.axis_index("s") * NC + lax.axis_index("c")
    ci0 = wid * NCHUNK

    iota = lax.iota(jnp.int32, L)

    def start_in(ci, s):
        pltpu.async_copy(
            tt_hbm.at[:, pl.ds(ci * 256, 256)], in_v.at[s], isems.at[s]
        )

    def wait_in(s):
        pltpu.make_async_copy(
            tt_hbm.at[:, pl.ds(0, 256)], in_v.at[s], isems.at[s]
        ).wait()

    def start_out(ci, s):
        pltpu.async_copy(
            out_v.at[s], t2_hbm.at[pl.ds(ci * 128, 128)], osems.at[s]
        )

    def wait_out(s):
        pltpu.make_async_copy(
            out_v.at[s], t2_hbm.at[pl.ds(0, 128)], osems.at[s]
        ).wait()

    def transpose(inp, out, npg):
        @plsc.parallel_loop(0, DIM, unroll=4)
        def _(dd):
            d = lax.bitwise_and(dd + iota, 63)
            for pg in range(npg):
                p = iota + L * pg
                for half in range(2):
                    val = plsc.load_gather(inp, [d, 2 * p + half])
                    plsc.store_scatter(out, [p, half * DIM + d], val)

    def step(i, first, last):
        s = i % 2
        wait_in(s)
        if not first:
            wait_out(s)
        transpose(in_v.at[s], out_v.at[s], 8)
        start_out(ci0 + i, s)
        if not last:
            start_in(ci0 + i + 2, s)

    start_in(ci0, 0)
    start_in(ci0 + 1, 1)
    step(0, first=True, last=False)
    step(1, first=True, last=False)

    @pl.loop(2, NCHUNK - 2)
    def _(i):
        step(i, first=False, last=False)

    step(NCHUNK - 2, first=False, last=True)
    step(NCHUNK - 1, first=False, last=True)
    wait_out(0)
    wait_out(1)

    for k in range(2):
        @pl.when(wid == k)
        def _():
            pltpu.sync_copy(tt_hbm.at[:, pl.ds((NMAIN + k) * 256, 256)], in_v.at[0])
            transpose(in_v.at[0], out_v.at[0], 8)
            pltpu.sync_copy(out_v.at[0], t2_hbm.at[pl.ds((NMAIN + k) * 128, 128)])

    @pl.when(wid == 2)
    def _():
        pltpu.sync_copy(tt_hbm.at[:, pl.ds((NMAIN + 2) * 256, DIM)], tin_v)
        transpose(tin_v, tout_v, 2)
        pltpu.sync_copy(tout_v, t2_hbm.at[pl.ds((NMAIN + 2) * 128, 32)])


_repack = pl.kernel(
    _repack_body,
    out_type=jax.ShapeDtypeStruct((500000, 128), jnp.float32),
    mesh=plsc.VectorSubcoreMesh(
        core_axis_name="c", subcore_axis_name="s", num_cores=NC, num_subcores=NS
    ),
    scratch_types=[
        pltpu.VMEM((2, DIM, 256), jnp.float32),
        pltpu.VMEM((2, 128, 128), jnp.float32),
        pltpu.VMEM((DIM, DIM), jnp.float32),
        pltpu.VMEM((32, 128), jnp.float32),
        pltpu.SemaphoreType.DMA((2,)),
        pltpu.SemaphoreType.DMA((2,)),
    ],
    compiler_params=pltpu.CompilerParams(needs_layout_passes=False),
)


def kernel(x, table):
    table2 = _repack(table.T)
    xt = x.T
    outt = _gather(table2, xt)
    return outt.transpose(2, 0, 1)


# speedup vs baseline: 2.9403x; 1.0049x over previous
import jax
import jax.numpy as jnp
from jax import lax
from jax.experimental import pallas as pl
from jax.experimental.pallas import tpu as pltpu
from jax.experimental.pallas import tpu_sc as plsc

NC, NS = 2, 16
NW = NC * NS
CB = 128
NJ = 200
NG = 4
NO = 3
BATCH = 4096
HIST = 200
DIM = 64
L = 16


def _body(table2_hbm, xt_hbm, out_hbm, idx_v, pidx_v, rows_v, outb_v, gsems, wsems):
    wid = lax.axis_index("s") * NC + lax.axis_index("c")
    b0 = wid * CB

    pltpu.sync_copy(xt_hbm.at[:, pl.ds(b0, CB)], idx_v)

    iota = lax.iota(jnp.int32, L)
    iotas = [iota + (L * g) for g in range(8)]

    def start_gather(h, s):
        for g in range(8):
            pidx_v[s, pl.ds(L * g, L)] = lax.shift_right_logical(
                idx_v[h, pl.ds(L * g, L)], 1
            )
        pltpu.async_copy(
            table2_hbm.at[pidx_v.at[s]], rows_v.at[s], gsems.at[s]
        )

    def wait_gather(s):
        pltpu.make_async_copy(
            table2_hbm.at[pidx_v.at[s]], rows_v.at[s], gsems.at[s]
        ).wait()

    def start_write(h, o):
        pltpu.async_copy(
            outb_v.at[o], out_hbm.at[h, :, pl.ds(b0, CB)], wsems.at[o]
        )

    def wait_write(o):
        pltpu.make_async_copy(
            outb_v.at[o], out_hbm.at[0, :, pl.ds(b0, CB)], wsems.at[o]
        ).wait()

    def transpose_select(h, s, o):
        rows = rows_v.at[s]
        outb = outb_v.at[o]
        hvs = [
            lax.shift_left(
                lax.bitwise_and(idx_v[h, pl.ds(L * g, L)], 1), 6
            )
            for g in range(8)
        ]

        @plsc.parallel_loop(0, DIM, unroll=8)
        def _(d):
            w = lax.bitwise_and(d + iota, 63)
            for g in range(8):
                val = plsc.load_gather(rows, [iotas[g], hvs[g] + w])
                plsc.store_scatter(outb, [w, iotas[g]], val)

    def step(h, first, mid, last):
        s = h % NG
        if not last:
            start_gather(h + NG - 1, (h - 1) % NG)
        wait_gather(s)
        o = h % NO
        if not first:
            wait_write(o)
        transpose_select(h, s, o)
        start_write(h, o)

    for s in range(NG - 1):
        start_gather(s, s)

    NHEAD = max(NG - 1, NO)
    for h in range(NHEAD):
        step(h, first=True, mid=False, last=False)

    @pl.loop(NHEAD, NJ - NG + 1)
    def _(h):
        step(h, first=False, mid=True, last=False)

    for h in range(NJ - NG + 1, NJ):
        step(h, first=False, mid=False, last=True)

    for o in range(NO):
        wait_write(o)


_gather = pl.kernel(
    _body,
    out_type=jax.ShapeDtypeStruct((HIST, DIM, BATCH), jnp.float32),
    mesh=plsc.VectorSubcoreMesh(
        core_axis_name="c", subcore_axis_name="s", num_cores=NC, num_subcores=NS
    ),
    scratch_types=[
        pltpu.VMEM((NJ, CB), jnp.int32),
        pltpu.VMEM((NG, CB), jnp.int32),
        pltpu.VMEM((NG, CB, 128), jnp.float32),
        pltpu.VMEM((NO, DIM, CB), jnp.float32),
        pltpu.SemaphoreType.DMA((NG,)),
        pltpu.SemaphoreType.DMA((NO,)),
    ],
    compiler_params=pltpu.CompilerParams(needs_layout_passes=False),
)


NCHUNK = 122
NMAIN = NCHUNK * NW


def _repack_body(tt_hbm, t2_hbm, in_v, out_v, tin_v, tout_v, isems, osems):
    wid = lax.axis_index("s") * NC + lax.axis_index("c")
    ci0 = wid * NCHUNK

    iota = lax.iota(jnp.int32, L)

    def start_in(ci, s):
        pltpu.async_copy(
            tt_hbm.at[:, pl.ds(ci * 256, 256)], in_v.at[s], isems.at[s]
        )

    def wait_in(s):
        pltpu.make_async_copy(
            tt_hbm.at[:, pl.ds(0, 256)], in_v.at[s], isems.at[s]
        ).wait()

    def start_out(ci, s):
        pltpu.async_copy(
            out_v.at[s], t2_hbm.at[pl.ds(ci * 128, 128)], osems.at[s]
        )

    def wait_out(s):
        pltpu.make_async_copy(
            out_v.at[s], t2_hbm.at[pl.ds(0, 128)], osems.at[s]
        ).wait()

    def transpose(inp, out, npg):
        @plsc.parallel_loop(0, DIM, unroll=4)
        def _(dd):
            d = lax.bitwise_and(dd + iota, 63)
            for pg in range(npg):
                p = iota + L * pg
                for half in range(2):
                    val = plsc.load_gather(inp, [d, 2 * p + half])
                    plsc.store_scatter(out, [p, half * DIM + d], val)

    def step(i, first, last):
        s = i % 2
        wait_in(s)
        if not first:
            wait_out(s)
        transpose(in_v.at[s], out_v.at[s], 8)
        start_out(ci0 + i, s)
        if not last:
            start_in(ci0 + i + 2, s)

    start_in(ci0, 0)
    start_in(ci0 + 1, 1)
    step(0, first=True, last=False)
    step(1, first=True, last=False)

    @pl.loop(2, NCHUNK - 2)
    def _(i):
        step(i, first=False, last=False)

    step(NCHUNK - 2, first=False, last=True)
    step(NCHUNK - 1, first=False, last=True)
    wait_out(0)
    wait_out(1)

    for k in range(2):
        @pl.when(wid == k)
        def _():
            pltpu.sync_copy(tt_hbm.at[:, pl.ds((NMAIN + k) * 256, 256)], in_v.at[0])
            transpose(in_v.at[0], out_v.at[0], 8)
            pltpu.sync_copy(out_v.at[0], t2_hbm.at[pl.ds((NMAIN + k) * 128, 128)])

    @pl.when(wid == 2)
    def _():
        pltpu.sync_copy(tt_hbm.at[:, pl.ds((NMAIN + 2) * 256, DIM)], tin_v)
        transpose(tin_v, tout_v, 2)
        pltpu.sync_copy(tout_v, t2_hbm.at[pl.ds((NMAIN + 2) * 128, 32)])


_repack = pl.kernel(
    _repack_body,
    out_type=jax.ShapeDtypeStruct((500000, 128), jnp.float32),
    mesh=plsc.VectorSubcoreMesh(
        core_axis_name="c", subcore_axis_name="s", num_cores=NC, num_subcores=NS
    ),
    scratch_types=[
        pltpu.VMEM((2, DIM, 256), jnp.float32),
        pltpu.VMEM((2, 128, 128), jnp.float32),
        pltpu.VMEM((DIM, DIM), jnp.float32),
        pltpu.VMEM((32, 128), jnp.float32),
        pltpu.SemaphoreType.DMA((2,)),
        pltpu.SemaphoreType.DMA((2,)),
    ],
    compiler_params=pltpu.CompilerParams(needs_layout_passes=False),
)


def kernel(x, table):
    table2 = _repack(table.T)
    xt = x.T
    outt = _gather(table2, xt)
    return outt.transpose(2, 0, 1)
